# baseline (device time: 121566 ns/iter reference)
import jax
import jax.numpy as jnp
from jax import lax
from jax.experimental import pallas as pl
from jax.experimental.pallas import tpu as pltpu

N_DEV = 16
N_HOPS = 2 * (N_DEV - 1)
N_CPD = 2
N_RPD = 2
N_STREAMS = 2 * N_CPD * N_RPD
N_SLOTS = 4

_RING = (0, 4, 8, 12, 15, 11, 7, 3, 2, 6, 10, 14, 13, 9, 5, 1)
_POS = [0] * N_DEV
_NEXT = [0] * N_DEV
_PREV = [0] * N_DEV
for _p, _i in enumerate(_RING):
    _POS[_i] = _p
    _NEXT[_i] = _RING[(_p + 1) % N_DEV]
    _PREV[_i] = _RING[(_p - 1) % N_DEV]


def _sel(table, idx):
    out = jnp.int32(0)
    for i, v in enumerate(table):
        out = out + jnp.int32(v) * (idx == i).astype(jnp.int32)
    return out


def _gelu(z):
    return 0.5 * z * (1.0 + jnp.tanh(0.7978845608 * (z + 0.044715 * z * z * z)))


def kernel(A, B):
    m, k = A.shape
    k2, n = B.shape
    assert k == k2
    m_chunk = m // N_DEV
    m_sub = m_chunk // N_RPD
    n_sub = (n // 2) // N_CPD

    streams = []
    for c in range(N_CPD):
        for r in range(N_RPD):
            streams.append((True, c * n_sub, r * m_sub))
            streams.append((False, n // 2 + c * n_sub, r * m_sub))

    def body(a_ref, b_ref, out_ref, send_buf, recv_buf, send_sems, recv_sems,
             credit_sems):
        my = lax.axis_index("i")
        pos = _sel(_POS, my)
        nxt = _sel(_NEXT, my)
        prv = _sel(_PREV, my)

        barrier_sem = pltpu.get_barrier_semaphore()
        for nbr in (prv, nxt):
            pl.semaphore_signal(
                barrier_sem, inc=1,
                device_id=(nbr,), device_id_type=pl.DeviceIdType.MESH,
            )
        pl.semaphore_wait(barrier_sem, 2)

        def compute_chunk(j):
            out_ref[pl.ds(j * m_chunk, m_chunk), :] = jnp.dot(
                a_ref[pl.ds(j * m_chunk, m_chunk), :], b_ref[:, :],
                preferred_element_type=jnp.float32,
            )

        def tile(kk, j):
            fwd, col_off, row_off = streams[kk]
            return (pl.ds(j * m_chunk + row_off, m_sub),
                    pl.ds(col_off, n_sub))

        def dst(kk):
            return nxt if streams[kk][0] else prv

        def crd(kk):
            return prv if streams[kk][0] else nxt

        def rs_send_chunk(kk, s):
            if streams[kk][0]:
                return lax.rem(pos + N_DEV - s, N_DEV)
            return lax.rem(pos + s, N_DEV)

        def own_chunk(kk):
            if streams[kk][0]:
                return lax.rem(pos + 1, N_DEV)
            return lax.rem(pos + N_DEV - 1, N_DEV)

        def ag_recv_chunk(kk, s):
            if streams[kk][0]:
                return lax.rem(pos + N_DEV - s, N_DEV)
            return lax.rem(pos + s, N_DEV)

        def rdma(kk, h):
            slot = h % N_SLOTS
            return pltpu.make_async_remote_copy(
                src_ref=send_buf.at[kk, slot],
                dst_ref=recv_buf.at[kk, slot],
                send_sem=send_sems.at[kk, slot],
                recv_sem=recv_sems.at[kk, slot],
                device_id=(dst(kk),),
                device_id_type=pl.DeviceIdType.MESH,
            )

        def fwd(kk, h):
            return pltpu.make_async_remote_copy(
                src_ref=recv_buf.at[kk, (h - 1) % N_SLOTS],
                dst_ref=recv_buf.at[kk, h % N_SLOTS],
                send_sem=send_sems.at[kk, h % N_SLOTS],
                recv_sem=recv_sems.at[kk, h % N_SLOTS],
                device_id=(dst(kk),),
                device_id_type=pl.DeviceIdType.MESH,
            )

        def credit(kk):
            pl.semaphore_signal(
                credit_sems.at[kk], inc=1,
                device_id=(crd(kk),), device_id_type=pl.DeviceIdType.MESH,
            )

        def stage(kk, h):
            slot = h % N_SLOTS
            if h == 0:
                r, c = tile(kk, rs_send_chunk(kk, 0))
                send_buf[kk, slot, :, :] = out_ref[r, c]
            elif h <= N_DEV - 2:
                r, c = tile(kk, rs_send_chunk(kk, h))
                send_buf[kk, slot, :, :] = (
                    recv_buf[kk, (h - 1) % N_SLOTS, :, :] + out_ref[r, c]
                )
            else:
                r, c = tile(kk, own_chunk(kk))
                red = recv_buf[kk, (h - 1) % N_SLOTS, :, :] + out_ref[r, c]
                g = _gelu(red)
                out_ref[r, c] = g
                send_buf[kk, slot, :, :] = g

        def store(kk, h):
            if h >= N_DEV - 1:
                s = h - (N_DEV - 1)
                r, c = tile(kk, ag_recv_chunk(kk, s))
                out_ref[r, c] = recv_buf[kk, h % N_SLOTS, :, :]

        order = range(N_STREAMS)

        compute_chunk(pos)
        for kk in order:
            stage(kk, 0)
            rdma(kk, 0).start()

        for h in range(1, N_DEV):
            if h <= N_DEV // 2:
                compute_chunk(lax.rem(pos + N_DEV - h, N_DEV))
                if h < N_DEV // 2:
                    compute_chunk(lax.rem(pos + h, N_DEV))
            for kk in order:
                if h >= N_SLOTS:
                    rdma(kk, h).wait_send()
                rdma(kk, h - 1).wait_recv()
                stage(kk, h)
                credit(kk)
                if h >= N_SLOTS:
                    pl.semaphore_wait(credit_sems.at[kk], 1)
                rdma(kk, h).start()

        for h in range(N_DEV, N_HOPS):
            for kk in order:
                f = fwd(kk, h)
                if h <= N_DEV - 1 + N_SLOTS:
                    f.wait_send()
                rdma(kk, h - 1).wait_recv()
                pl.semaphore_wait(credit_sems.at[kk], 1)
                f.start()
            for kk in order:
                store(kk, h - 1)
                fwd(kk, h).wait_send()
                credit(kk)

        for kk in order:
            rdma(kk, N_HOPS - 1).wait_recv()
            store(kk, N_HOPS - 1)
            credit(kk)
            pl.semaphore_wait(credit_sems.at[kk], N_SLOTS)

    m_sub_ = m // N_DEV // N_RPD
    n_sub_ = n // 2 // N_CPD
    return pl.pallas_call(
        body,
        out_shape=jax.ShapeDtypeStruct((m, n), jnp.float32),
        in_specs=[
            pl.BlockSpec(memory_space=pltpu.VMEM),
            pl.BlockSpec(memory_space=pltpu.VMEM),
        ],
        out_specs=pl.BlockSpec(memory_space=pltpu.VMEM),
        scratch_shapes=[
            pltpu.VMEM((N_STREAMS, N_SLOTS, m_sub_, n_sub_), jnp.float32),
            pltpu.VMEM((N_STREAMS, N_SLOTS, m_sub_, n_sub_), jnp.float32),
            pltpu.SemaphoreType.DMA((N_STREAMS, N_SLOTS)),
            pltpu.SemaphoreType.DMA((N_STREAMS, N_SLOTS)),
            pltpu.SemaphoreType.REGULAR((N_STREAMS,)),
        ],
        compiler_params=pltpu.CompilerParams(collective_id=0),
    )(A, B)


# device time: 114953 ns/iter; 1.0575x vs baseline; 1.0575x over previous
import jax
import jax.numpy as jnp
from jax import lax
from jax.experimental import pallas as pl
from jax.experimental.pallas import tpu as pltpu

N_DEV = 16
N_HOPS = 2 * (N_DEV - 1)
N_CPD = 2
N_RPD = 2
N_STREAMS = 2 * N_CPD * N_RPD
N_SLOTS = 4

_RING = (0, 4, 8, 12, 15, 11, 7, 3, 2, 6, 10, 14, 13, 9, 5, 1)
_POS = [0] * N_DEV
_NEXT = [0] * N_DEV
_PREV = [0] * N_DEV
for _p, _i in enumerate(_RING):
    _POS[_i] = _p
    _NEXT[_i] = _RING[(_p + 1) % N_DEV]
    _PREV[_i] = _RING[(_p - 1) % N_DEV]


def _sel(table, idx):
    out = jnp.int32(0)
    for i, v in enumerate(table):
        out = out + jnp.int32(v) * (idx == i).astype(jnp.int32)
    return out


def _gelu(z):
    return 0.5 * z * (1.0 + jnp.tanh(0.7978845608 * (z + 0.044715 * z * z * z)))


def kernel(A, B):
    m, k = A.shape
    k2, n = B.shape
    assert k == k2
    m_chunk = m // N_DEV
    m_sub = m_chunk // N_RPD
    n_sub = (n // 2) // N_CPD

    streams = []
    for c in range(N_CPD):
        for r in range(N_RPD):
            streams.append((True, c * n_sub, r * m_sub))
            streams.append((False, n // 2 + c * n_sub, r * m_sub))

    def body(a_ref, b_ref, out_ref, send_buf, recv_buf, send_sems, recv_sems,
             credit_sems):
        my = lax.axis_index("i")
        pos = _sel(_POS, my)
        nxt = _sel(_NEXT, my)
        prv = _sel(_PREV, my)

        barrier_sem = pltpu.get_barrier_semaphore()
        for nbr in (prv, nxt):
            pl.semaphore_signal(
                barrier_sem, inc=1,
                device_id=(nbr,), device_id_type=pl.DeviceIdType.MESH,
            )
        pl.semaphore_wait(barrier_sem, 2)

        def compute_chunk(j):
            out_ref[pl.ds(j * m_chunk, m_chunk), :] = jnp.dot(
                a_ref[pl.ds(j * m_chunk, m_chunk), :], b_ref[:, :],
                preferred_element_type=jnp.float32,
            )

        def tile(kk, j):
            fwd, col_off, row_off = streams[kk]
            return (pl.ds(j * m_chunk + row_off, m_sub),
                    pl.ds(col_off, n_sub))

        def dst(kk):
            return nxt if streams[kk][0] else prv

        def crd(kk):
            return prv if streams[kk][0] else nxt

        def rs_send_chunk(kk, s):
            if streams[kk][0]:
                return lax.rem(pos + N_DEV - s, N_DEV)
            return lax.rem(pos + s, N_DEV)

        def own_chunk(kk):
            if streams[kk][0]:
                return lax.rem(pos + 1, N_DEV)
            return lax.rem(pos + N_DEV - 1, N_DEV)

        def ag_recv_chunk(kk, s):
            if streams[kk][0]:
                return lax.rem(pos + N_DEV - s, N_DEV)
            return lax.rem(pos + s, N_DEV)

        def rdma(kk, h):
            slot = h % N_SLOTS
            return pltpu.make_async_remote_copy(
                src_ref=send_buf.at[kk, slot],
                dst_ref=recv_buf.at[kk, slot],
                send_sem=send_sems.at[kk, slot],
                recv_sem=recv_sems.at[kk, slot],
                device_id=(dst(kk),),
                device_id_type=pl.DeviceIdType.MESH,
            )

        def fwd(kk, h):
            return pltpu.make_async_remote_copy(
                src_ref=recv_buf.at[kk, (h - 1) % N_SLOTS],
                dst_ref=recv_buf.at[kk, h % N_SLOTS],
                send_sem=send_sems.at[kk, h % N_SLOTS],
                recv_sem=recv_sems.at[kk, h % N_SLOTS],
                device_id=(dst(kk),),
                device_id_type=pl.DeviceIdType.MESH,
            )

        def credit(kk):
            pl.semaphore_signal(
                credit_sems.at[kk], inc=1,
                device_id=(crd(kk),), device_id_type=pl.DeviceIdType.MESH,
            )

        def stage(kk, h):
            slot = h % N_SLOTS
            if h == 0:
                r, c = tile(kk, rs_send_chunk(kk, 0))
                send_buf[kk, slot, :, :] = out_ref[r, c]
            elif h <= N_DEV - 2:
                r, c = tile(kk, rs_send_chunk(kk, h))
                send_buf[kk, slot, :, :] = (
                    recv_buf[kk, (h - 1) % N_SLOTS, :, :] + out_ref[r, c]
                )
            else:
                r, c = tile(kk, own_chunk(kk))
                red = recv_buf[kk, (h - 1) % N_SLOTS, :, :] + out_ref[r, c]
                g = _gelu(red)
                out_ref[r, c] = g
                send_buf[kk, slot, :, :] = g

        def store(kk, h):
            if h >= N_DEV - 1:
                s = h - (N_DEV - 1)
                r, c = tile(kk, ag_recv_chunk(kk, s))
                out_ref[r, c] = recv_buf[kk, h % N_SLOTS, :, :]

        order = range(N_STREAMS)

        compute_chunk(pos)
        for kk in order:
            stage(kk, 0)
            rdma(kk, 0).start()

        for h in range(1, N_DEV):
            if h <= N_DEV // 2:
                compute_chunk(lax.rem(pos + N_DEV - h, N_DEV))
                if h < N_DEV // 2:
                    compute_chunk(lax.rem(pos + h, N_DEV))
            for kk in order:
                if h >= N_SLOTS:
                    rdma(kk, h).wait_send()
                rdma(kk, h - 1).wait_recv()
                stage(kk, h)
                credit(kk)
                if h >= N_SLOTS:
                    pl.semaphore_wait(credit_sems.at[kk], 1)
                rdma(kk, h).start()

        for h in range(N_DEV, N_HOPS):
            for kk in order:
                f = fwd(kk, h)
                if h <= N_DEV - 1 + N_SLOTS:
                    f.wait_send()
                rdma(kk, h - 1).wait_recv()
                pl.semaphore_wait(credit_sems.at[kk], 1)
                f.start()
            if h > N_DEV:
                for kk in order:
                    store(kk, h - 2)
                    fwd(kk, h - 1).wait_send()
                    credit(kk)

        for kk in order:
            store(kk, N_HOPS - 2)
            fwd(kk, N_HOPS - 1).wait_send()
            credit(kk)
        for kk in order:
            rdma(kk, N_HOPS - 1).wait_recv()
            store(kk, N_HOPS - 1)
            credit(kk)
            pl.semaphore_wait(credit_sems.at[kk], N_SLOTS)

    m_sub_ = m // N_DEV // N_RPD
    n_sub_ = n // 2 // N_CPD
    return pl.pallas_call(
        body,
        out_shape=jax.ShapeDtypeStruct((m, n), jnp.float32),
        in_specs=[
            pl.BlockSpec(memory_space=pltpu.VMEM),
            pl.BlockSpec(memory_space=pltpu.VMEM),
        ],
        out_specs=pl.BlockSpec(memory_space=pltpu.VMEM),
        scratch_shapes=[
            pltpu.VMEM((N_STREAMS, N_SLOTS, m_sub_, n_sub_), jnp.float32),
            pltpu.VMEM((N_STREAMS, N_SLOTS, m_sub_, n_sub_), jnp.float32),
            pltpu.SemaphoreType.DMA((N_STREAMS, N_SLOTS)),
            pltpu.SemaphoreType.DMA((N_STREAMS, N_SLOTS)),
            pltpu.SemaphoreType.REGULAR((N_STREAMS,)),
        ],
        compiler_params=pltpu.CompilerParams(collective_id=0),
    )(A, B)


# device time: 83233 ns/iter; 1.4606x vs baseline; 1.3811x over previous
import jax
import jax.numpy as jnp
from jax import lax
from jax.experimental import pallas as pl
from jax.experimental.pallas import tpu as pltpu

N_DEV = 16
N_HOPS = 2 * (N_DEV - 1)
N_CPD = 2
N_RPD = 2
N_STREAMS = 2 * N_CPD * N_RPD
N_SLOTS = 4

_RING = (0, 4, 8, 12, 15, 11, 7, 3, 2, 6, 10, 14, 13, 9, 5, 1)
_POS = [0] * N_DEV
_NEXT = [0] * N_DEV
_PREV = [0] * N_DEV
for _p, _i in enumerate(_RING):
    _POS[_i] = _p
    _NEXT[_i] = _RING[(_p + 1) % N_DEV]
    _PREV[_i] = _RING[(_p - 1) % N_DEV]


def _sel(table, idx):
    out = jnp.int32(0)
    for i, v in enumerate(table):
        out = out + jnp.int32(v) * (idx == i).astype(jnp.int32)
    return out


def _gelu(z):
    return 0.5 * z * (1.0 + jnp.tanh(0.7978845608 * (z + 0.044715 * z * z * z)))


def kernel(A, B):
    m, k = A.shape
    k2, n = B.shape
    assert k == k2
    m_chunk = m // N_DEV
    m_sub = m_chunk // N_RPD
    n_sub = (n // 2) // N_CPD

    streams = []
    for c in range(N_CPD):
        for r in range(N_RPD):
            streams.append((True, c * n_sub, r * m_sub))
            streams.append((False, n // 2 + c * n_sub, r * m_sub))

    def body(a_ref, b_ref, out_ref, send_buf, recv_buf, b_bf, send_sems,
             recv_sems, credit_sems):
        my = lax.axis_index("i")
        pos = _sel(_POS, my)
        nxt = _sel(_NEXT, my)
        prv = _sel(_PREV, my)

        barrier_sem = pltpu.get_barrier_semaphore()
        for nbr in (prv, nxt):
            pl.semaphore_signal(
                barrier_sem, inc=1,
                device_id=(nbr,), device_id_type=pl.DeviceIdType.MESH,
            )
        pl.semaphore_wait(barrier_sem, 2)

        b_bf[:, :] = b_ref[:, :].astype(jnp.bfloat16)

        def compute_chunk(j):
            out_ref[pl.ds(j * m_chunk, m_chunk), :] = jnp.dot(
                a_ref[pl.ds(j * m_chunk, m_chunk), :].astype(jnp.bfloat16),
                b_bf[:, :],
                preferred_element_type=jnp.float32,
            )

        def tile(kk, j):
            fwd, col_off, row_off = streams[kk]
            return (pl.ds(j * m_chunk + row_off, m_sub),
                    pl.ds(col_off, n_sub))

        def dst(kk):
            return nxt if streams[kk][0] else prv

        def crd(kk):
            return prv if streams[kk][0] else nxt

        def rs_send_chunk(kk, s):
            if streams[kk][0]:
                return lax.rem(pos + N_DEV - s, N_DEV)
            return lax.rem(pos + s, N_DEV)

        def own_chunk(kk):
            if streams[kk][0]:
                return lax.rem(pos + 1, N_DEV)
            return lax.rem(pos + N_DEV - 1, N_DEV)

        def ag_recv_chunk(kk, s):
            if streams[kk][0]:
                return lax.rem(pos + N_DEV - s, N_DEV)
            return lax.rem(pos + s, N_DEV)

        def rdma(kk, h):
            slot = h % N_SLOTS
            return pltpu.make_async_remote_copy(
                src_ref=send_buf.at[kk, slot],
                dst_ref=recv_buf.at[kk, slot],
                send_sem=send_sems.at[kk, slot],
                recv_sem=recv_sems.at[kk, slot],
                device_id=(dst(kk),),
                device_id_type=pl.DeviceIdType.MESH,
            )

        def fwd(kk, h):
            return pltpu.make_async_remote_copy(
                src_ref=recv_buf.at[kk, (h - 1) % N_SLOTS],
                dst_ref=recv_buf.at[kk, h % N_SLOTS],
                send_sem=send_sems.at[kk, h % N_SLOTS],
                recv_sem=recv_sems.at[kk, h % N_SLOTS],
                device_id=(dst(kk),),
                device_id_type=pl.DeviceIdType.MESH,
            )

        def credit(kk):
            pl.semaphore_signal(
                credit_sems.at[kk], inc=1,
                device_id=(crd(kk),), device_id_type=pl.DeviceIdType.MESH,
            )

        def stage(kk, h):
            slot = h % N_SLOTS
            if h == 0:
                r, c = tile(kk, rs_send_chunk(kk, 0))
                send_buf[kk, slot, :, :] = out_ref[r, c].astype(jnp.bfloat16)
            elif h <= N_DEV - 2:
                r, c = tile(kk, rs_send_chunk(kk, h))
                send_buf[kk, slot, :, :] = (
                    recv_buf[kk, (h - 1) % N_SLOTS, :, :].astype(jnp.float32)
                    + out_ref[r, c]
                ).astype(jnp.bfloat16)
            else:
                r, c = tile(kk, own_chunk(kk))
                red = (recv_buf[kk, (h - 1) % N_SLOTS, :, :]
                       .astype(jnp.float32) + out_ref[r, c])
                g = _gelu(red)
                out_ref[r, c] = g
                send_buf[kk, slot, :, :] = g.astype(jnp.bfloat16)

        def store(kk, h):
            if h >= N_DEV - 1:
                s = h - (N_DEV - 1)
                r, c = tile(kk, ag_recv_chunk(kk, s))
                out_ref[r, c] = recv_buf[kk, h % N_SLOTS, :, :].astype(jnp.float32)

        order = range(N_STREAMS)

        compute_chunk(pos)
        for kk in order:
            stage(kk, 0)
            rdma(kk, 0).start()

        for h in range(1, N_DEV):
            if h <= N_DEV // 2:
                compute_chunk(lax.rem(pos + N_DEV - h, N_DEV))
                if h < N_DEV // 2:
                    compute_chunk(lax.rem(pos + h, N_DEV))
            for kk in order:
                if h >= N_SLOTS:
                    rdma(kk, h).wait_send()
                rdma(kk, h - 1).wait_recv()
                stage(kk, h)
                credit(kk)
                if h >= N_SLOTS:
                    pl.semaphore_wait(credit_sems.at[kk], 1)
                rdma(kk, h).start()

        for h in range(N_DEV, N_HOPS):
            for kk in order:
                f = fwd(kk, h)
                if h <= N_DEV - 1 + N_SLOTS:
                    f.wait_send()
                rdma(kk, h - 1).wait_recv()
                pl.semaphore_wait(credit_sems.at[kk], 1)
                f.start()
            if h > N_DEV:
                for kk in order:
                    store(kk, h - 2)
                    fwd(kk, h - 1).wait_send()
                    credit(kk)

        for kk in order:
            store(kk, N_HOPS - 2)
            fwd(kk, N_HOPS - 1).wait_send()
            credit(kk)
        for kk in order:
            rdma(kk, N_HOPS - 1).wait_recv()
            store(kk, N_HOPS - 1)
            credit(kk)
            pl.semaphore_wait(credit_sems.at[kk], N_SLOTS)

    m_sub_ = m // N_DEV // N_RPD
    n_sub_ = n // 2 // N_CPD
    return pl.pallas_call(
        body,
        out_shape=jax.ShapeDtypeStruct((m, n), jnp.float32),
        in_specs=[
            pl.BlockSpec(memory_space=pltpu.VMEM),
            pl.BlockSpec(memory_space=pltpu.VMEM),
        ],
        out_specs=pl.BlockSpec(memory_space=pltpu.VMEM),
        scratch_shapes=[
            pltpu.VMEM((N_STREAMS, N_SLOTS, m_sub_, n_sub_), jnp.bfloat16),
            pltpu.VMEM((N_STREAMS, N_SLOTS, m_sub_, n_sub_), jnp.bfloat16),
            pltpu.VMEM((k, n), jnp.bfloat16),
            pltpu.SemaphoreType.DMA((N_STREAMS, N_SLOTS)),
            pltpu.SemaphoreType.DMA((N_STREAMS, N_SLOTS)),
            pltpu.SemaphoreType.REGULAR((N_STREAMS,)),
        ],
        compiler_params=pltpu.CompilerParams(collective_id=0),
    )(A, B)
